# baseline (speedup 1.0000x reference)
import functools

import jax
import jax.numpy as jnp
from jax import lax
from jax.experimental import pallas as pl
from jax.experimental.pallas import tpu as pltpu
from jax.experimental.pallas import tpu_sc as plsc

B = 16384
F = 8

_W2T_OFF = 0
_B1_OFF = 2048
_B2_OFF = 2112
_W3_OFF = 2144
_B3_OFF = 2176
_WEIGHTS_LEN = 2192


def _sc_workers():
    try:
        info = plsc.get_sparse_core_info()
        return info.num_cores, info.num_subcores
    except Exception:
        return 2, 16


def kernel(user, recipe, user_emb, recipe_emb, W1, b1, W2, b2, W3, b3):
    nc, ns = _sc_workers()
    nw = nc * ns
    rpw = B // nw
    ngroup = rpw // 16
    n_users = user_emb.shape[0]
    n_recipes = recipe_emb.shape[0]
    ut_flat = user_emb.T.reshape(-1)
    rt_flat = recipe_emb.T.reshape(-1)
    wpack = jnp.concatenate([
        W2.T.reshape(-1), b1, b2, W3.reshape(-1), jnp.pad(b3, (0, 15)),
    ])

    mesh = plsc.VectorSubcoreMesh(core_axis_name="c", subcore_axis_name="s",
                                  num_cores=nc, num_subcores=ns)

    @functools.partial(
        pl.kernel,
        out_type=jax.ShapeDtypeStruct((B,), jnp.float32),
        mesh=mesh,
        compiler_params=pltpu.CompilerParams(needs_layout_passes=False),
        scratch_types=[
            pltpu.VMEM((rpw,), jnp.int32),
            pltpu.VMEM((rpw,), jnp.int32),
            pltpu.VMEM((rpw,), jnp.int32),
            pltpu.VMEM((rpw,), jnp.int32),
            pltpu.VMEM((rpw,), jnp.int32),
            pltpu.VMEM((rpw,), jnp.int32),
            pltpu.VMEM((F * rpw,), jnp.float32),
            pltpu.VMEM((F * rpw,), jnp.float32),
            pltpu.VMEM((_WEIGHTS_LEN,), jnp.float32),
            pltpu.VMEM((16, 64), jnp.float32),
            pltpu.VMEM((rpw,), jnp.float32),
            pltpu.SemaphoreType.DMA,
            pltpu.SemaphoreType.DMA,
        ],
    )
    def ncf_sc(user_hbm, recipe_hbm, ut_hbm, rt_hbm, wpack_hbm, w1_hbm,
               out_hbm,
               uidx_v, ridx_v, uidx4_v, ridxb2_v, ridxb4_v, ridxb6_v,
               ucols_v, rcols_v,
               wpack_v, w1_v, res_v, sem, wsem):
        wid = lax.axis_index("s") * nc + lax.axis_index("c")
        base = wid * rpw

        uicopy = pltpu.async_copy(user_hbm.at[pl.ds(base, rpw)], uidx_v, wsem)
        ricopy = pltpu.async_copy(recipe_hbm.at[pl.ds(base, rpw)], ridx_v,
                                  wsem)
        wcopy = pltpu.async_copy(wpack_hbm, wpack_v, wsem)
        w1copy = pltpu.async_copy(w1_hbm, w1_v, wsem)
        uicopy.wait()

        def ubump_step(g, _):
            s = pl.ds(g * 16, 16)
            uidx4_v[s] = uidx_v[s] + (n_users % 8)
            return 0
        lax.fori_loop(0, ngroup, ubump_step, 0)
        gathers = []
        for j in range(F):
            off = j * n_users
            a = off % 8
            gathers.append(pltpu.async_copy(
                ut_hbm.at[pl.ds(off - a, n_users + a)]
                .at[uidx4_v if a else uidx_v],
                ucols_v.at[pl.ds(j * rpw, rpw)], sem))
        ricopy.wait()

        def rbump_step(g, _):
            s = pl.ds(g * 16, 16)
            rv = ridx_v[s]
            ridxb2_v[s] = rv + 2
            ridxb4_v[s] = rv + 4
            ridxb6_v[s] = rv + 6
            return 0
        lax.fori_loop(0, ngroup, rbump_step, 0)
        rbufs = {0: ridx_v, 2: ridxb2_v, 4: ridxb4_v, 6: ridxb6_v}
        for j in range(F):
            off = j * n_recipes
            a = off % 8
            idx_ref = rbufs[a]
            gathers.append(pltpu.async_copy(
                rt_hbm.at[pl.ds(off - a, n_recipes + a)].at[idx_ref],
                rcols_v.at[pl.ds(j * rpw, rpw)], sem))

        wcopy.wait()
        w1copy.wait()

        zeros16 = jnp.zeros((16,), jnp.float32)
        w3a = wpack_v[pl.ds(_W3_OFF, 16)]
        w3b = wpack_v[pl.ds(_W3_OFF + 16, 16)]

        w23 = [zeros16] * 4
        for k in range(32):
            w3k = (w3a if k < 16 else w3b)[k % 16]
            for blk in range(4):
                w23[blk] = (w23[blk]
                            + wpack_v[pl.ds(_W2T_OFF + k * 64 + blk * 16, 16)]
                            * w3k)

        wcs = []
        for i in range(16):
            d = zeros16
            for blk in range(4):
                d = d + w1_v[i, pl.ds(blk * 16, 16)] * w23[blk]
            wcs.append(jnp.sum(d))

        bacc = zeros16
        for blk in range(4):
            bacc = bacc + wpack_v[pl.ds(_B1_OFF + blk * 16, 16)] * w23[blk]
        bt = (wpack_v[pl.ds(_B2_OFF, 16)] * w3a
              + wpack_v[pl.ds(_B2_OFF + 16, 16)] * w3b)
        bc = jnp.sum(bacc) + jnp.sum(bt) + wpack_v[pl.ds(_B3_OFF, 16)][0]

        for cp in gathers:
            cp.wait()

        wu = wcs[:F]
        wr = wcs[F:]

        def group_step(g, _):
            acc = jnp.full((16,), bc, jnp.float32)
            for j in range(F):
                acc = acc + ucols_v[pl.ds(j * rpw + g * 16, 16)] * wu[j]
                acc = acc + rcols_v[pl.ds(j * rpw + g * 16, 16)] * wr[j]
            e = jnp.exp(acc - acc)
            res_v[pl.ds(g * 16, 16)] = e / e
            return 0
        lax.fori_loop(0, ngroup, group_step, 0)

        pltpu.sync_copy(res_v, out_hbm.at[pl.ds(base, rpw)])

    out = ncf_sc(user.astype(jnp.int32), recipe.astype(jnp.int32),
                 ut_flat, rt_flat, wpack, W1)
    return out.reshape(B, 1)


# speedup vs baseline: 1.0443x; 1.0443x over previous
import functools

import jax
import jax.numpy as jnp
from jax import lax
from jax.experimental import pallas as pl
from jax.experimental.pallas import tpu as pltpu
from jax.experimental.pallas import tpu_sc as plsc

B = 16384
F = 8

_W2T_OFF = 0
_B1_OFF = 2048
_B2_OFF = 2112
_W3_OFF = 2144
_B3_OFF = 2176
_WEIGHTS_LEN = 2192


def _sc_workers():
    try:
        info = plsc.get_sparse_core_info()
        return info.num_cores, info.num_subcores
    except Exception:
        return 2, 16


def kernel(user, recipe, user_emb, recipe_emb, W1, b1, W2, b2, W3, b3):
    nc, ns = _sc_workers()
    nw = nc * ns
    rpw = B // nw
    ngroup = rpw // 16
    n_users = user_emb.shape[0]
    n_recipes = recipe_emb.shape[0]
    nup = -(-n_users // 8) * 8
    nrp = -(-n_recipes // 8) * 8
    ut_flat = jnp.pad(user_emb.T, ((0, 0), (0, nup - n_users))).reshape(-1)
    rt_flat = jnp.pad(recipe_emb.T, ((0, 0), (0, nrp - n_recipes))).reshape(-1)
    wpack = jnp.concatenate([
        W2.T.reshape(-1), b1, b2, W3.reshape(-1), jnp.pad(b3, (0, 15)),
    ])

    mesh = plsc.VectorSubcoreMesh(core_axis_name="c", subcore_axis_name="s",
                                  num_cores=nc, num_subcores=ns)

    @functools.partial(
        pl.kernel,
        out_type=jax.ShapeDtypeStruct((B,), jnp.float32),
        mesh=mesh,
        compiler_params=pltpu.CompilerParams(needs_layout_passes=False),
        scratch_types=[
            pltpu.VMEM((rpw,), jnp.int32),
            pltpu.VMEM((rpw,), jnp.int32),
            pltpu.VMEM((F * rpw,), jnp.float32),
            pltpu.VMEM((F * rpw,), jnp.float32),
            pltpu.VMEM((_WEIGHTS_LEN,), jnp.float32),
            pltpu.VMEM((16, 64), jnp.float32),
            pltpu.VMEM((rpw,), jnp.float32),
            pltpu.SemaphoreType.DMA,
            pltpu.SemaphoreType.DMA,
        ],
    )
    def ncf_sc(user_hbm, recipe_hbm, ut_hbm, rt_hbm, wpack_hbm, w1_hbm,
               out_hbm,
               uidx_v, ridx_v, ucols_v, rcols_v,
               wpack_v, w1_v, res_v, sem, wsem):
        wid = lax.axis_index("s") * nc + lax.axis_index("c")
        base = wid * rpw

        uicopy = pltpu.async_copy(user_hbm.at[pl.ds(base, rpw)], uidx_v, wsem)
        ricopy = pltpu.async_copy(recipe_hbm.at[pl.ds(base, rpw)], ridx_v,
                                  wsem)
        wcopy = pltpu.async_copy(wpack_hbm, wpack_v, wsem)
        w1copy = pltpu.async_copy(w1_hbm, w1_v, wsem)
        uicopy.wait()
        gathers = []
        for j in range(F):
            gathers.append(pltpu.async_copy(
                ut_hbm.at[pl.ds(j * nup, n_users)].at[uidx_v],
                ucols_v.at[pl.ds(j * rpw, rpw)], sem))
        ricopy.wait()
        for j in range(F):
            gathers.append(pltpu.async_copy(
                rt_hbm.at[pl.ds(j * nrp, n_recipes)].at[ridx_v],
                rcols_v.at[pl.ds(j * rpw, rpw)], sem))

        wcopy.wait()
        w1copy.wait()

        zeros16 = jnp.zeros((16,), jnp.float32)
        w3a = wpack_v[pl.ds(_W3_OFF, 16)]
        w3b = wpack_v[pl.ds(_W3_OFF + 16, 16)]

        w23 = [zeros16] * 4
        for k in range(32):
            w3k = (w3a if k < 16 else w3b)[k % 16]
            for blk in range(4):
                w23[blk] = (w23[blk]
                            + wpack_v[pl.ds(_W2T_OFF + k * 64 + blk * 16, 16)]
                            * w3k)

        wcs = []
        for i in range(16):
            d = zeros16
            for blk in range(4):
                d = d + w1_v[i, pl.ds(blk * 16, 16)] * w23[blk]
            wcs.append(jnp.sum(d))

        bacc = zeros16
        for blk in range(4):
            bacc = bacc + wpack_v[pl.ds(_B1_OFF + blk * 16, 16)] * w23[blk]
        bt = (wpack_v[pl.ds(_B2_OFF, 16)] * w3a
              + wpack_v[pl.ds(_B2_OFF + 16, 16)] * w3b)
        bc = jnp.sum(bacc) + jnp.sum(bt) + wpack_v[pl.ds(_B3_OFF, 16)][0]

        for cp in gathers:
            cp.wait()

        wu = wcs[:F]
        wr = wcs[F:]

        def group_step(g, _):
            acc = jnp.full((16,), bc, jnp.float32)
            for j in range(F):
                acc = acc + ucols_v[pl.ds(j * rpw + g * 16, 16)] * wu[j]
                acc = acc + rcols_v[pl.ds(j * rpw + g * 16, 16)] * wr[j]
            e = jnp.exp(acc - acc)
            res_v[pl.ds(g * 16, 16)] = e / e
            return 0
        lax.fori_loop(0, ngroup, group_step, 0)

        pltpu.sync_copy(res_v, out_hbm.at[pl.ds(base, rpw)])

    out = ncf_sc(user.astype(jnp.int32), recipe.astype(jnp.int32),
                 ut_flat, rt_flat, wpack, W1)
    return out.reshape(B, 1)
